# baseline (speedup 1.0000x reference)
import functools

import jax
import jax.numpy as jnp
from jax import lax
from jax.experimental import pallas as pl
from jax.experimental.pallas import tpu as pltpu
from jax.experimental.pallas import tpu_sc as plsc

NEMB = 8192
DIM = 32
RB = 1024
CB = 1024
NRB = NEMB // RB
NCB = NEMB // CB

_NC = 2
_NS = 16
_L = 16
_NW = _NC * _NS
_BPW = NEMB // _NW
_ICH = 128
_NCHUNK = _BPW // _ICH

MM_PRECISION = lax.Precision.DEFAULT


def _dist_body(x_ref, emb_ref, idx_ref, embn_ref, crow_ref):
    pid = pl.program_id(0)

    @pl.when(pid == 0)
    def _prep():
        emb = emb_ref[...]
        nrm = jnp.sqrt(jnp.sum(emb * emb, axis=1, keepdims=True))
        emb_n = emb / (nrm + 1e-4)
        embn_ref[...] = emb_n
        crow_ref[...] = lax.dot_general(
            jnp.ones((1, DIM), jnp.float32), emb_n * emb_n,
            (((1,), (1,)), ((), ())),
            preferred_element_type=jnp.float32,
            precision=lax.Precision.HIGHEST)

    x = x_ref[...]
    x2 = jnp.sum(x * x, axis=1, keepdims=True)

    best_val = jnp.full((RB, 1), jnp.inf, dtype=jnp.float32)
    best_idx = jnp.zeros((RB, 1), dtype=jnp.int32)
    for k in range(NCB):
        e_chunk = embn_ref[k * CB:(k + 1) * CB, :]
        mm = lax.dot_general(x, e_chunk, (((1,), (1,)), ((), ())),
                             preferred_element_type=jnp.float32,
                             precision=MM_PRECISION)
        d = (crow_ref[0:1, k * CB:(k + 1) * CB] + x2) - 2.0 * mm
        m = jnp.min(d, axis=1, keepdims=True)
        col = lax.broadcasted_iota(jnp.int32, (RB, CB), 1)
        idx = jnp.min(jnp.where(d == m, col, NEMB), axis=1,
                      keepdims=True) + k * CB
        upd = m < best_val
        best_val = jnp.where(upd, m, best_val)
        best_idx = jnp.where(upd, idx, best_idx)

    idx_ref[...] = best_idx


def _dist(x_flat, embedding):
    nrows = x_flat.shape[0]
    return pl.pallas_call(
        _dist_body,
        grid=(nrows // RB,),
        in_specs=[
            pl.BlockSpec((RB, DIM), lambda i: (i, 0)),
            pl.BlockSpec((NEMB, DIM), lambda i: (0, 0)),
        ],
        out_specs=pl.BlockSpec((RB, 1), lambda i: (i, 0)),
        out_shape=jax.ShapeDtypeStruct((nrows, 1), jnp.int32),
        scratch_shapes=[
            pltpu.VMEM((NEMB, DIM), jnp.float32),
            pltpu.VMEM((1, NEMB), jnp.float32),
        ],
        compiler_params=pltpu.CompilerParams(
            dimension_semantics=("arbitrary",)),
    )(x_flat, embedding)


DPAD = 128


def _sc_gather_hist(emb_pad, idx3d):
    nchunk = idx3d.shape[1]
    bpw = nchunk * _ICH
    nrows = _NW * bpw
    mesh = plsc.VectorSubcoreMesh(core_axis_name="c", subcore_axis_name="s")

    @functools.partial(
        pl.kernel,
        mesh=mesh,
        out_type=[
            jax.ShapeDtypeStruct((nrows, DPAD), jnp.float32),
            jax.ShapeDtypeStruct((NEMB,), jnp.float32),
            jax.ShapeDtypeStruct((NEMB,), jnp.float32),
        ],
        scratch_types=[
            pltpu.VMEM((nchunk, _ICH), jnp.int32),
            pltpu.VMEM((bpw, DPAD), jnp.float32),
            pltpu.VMEM((_ICH,), jnp.float32),
            pltpu.VMEM((NEMB,), jnp.float32),
            pltpu.VMEM_SHARED((NEMB,), jnp.float32),
            pltpu.SemaphoreType.DMA,
        ],
    )
    def k(emb_hbm, idx_hbm, q_hbm, hist0_hbm, hist1_hbm,
          idx_v, rows_v, ones_v, stage_v, sh_hist, sem):
        cc = lax.axis_index("c")
        ss = lax.axis_index("s")
        wid = ss * _NC + cc
        pltpu.sync_copy(idx_hbm.at[wid], idx_v)
        for j in range(nchunk):
            pltpu.async_copy(emb_hbm.at[idx_v.at[j]],
                             rows_v.at[pl.ds(j * _ICH, _ICH)], sem).wait()
        pltpu.sync_copy(rows_v, q_hbm.at[pl.ds(wid * bpw, bpw)])

        def _ones(i, carry):
            ones_v[pl.ds(i * _L, _L)] = jnp.ones((_L,), jnp.float32)
            return carry

        lax.fori_loop(0, _ICH // _L, _ones, 0)

        @pl.when(ss == 0)
        def _zero():
            def _zb(i, carry):
                stage_v[pl.ds(i * _L, _L)] = jnp.zeros((_L,), jnp.float32)
                return carry

            lax.fori_loop(0, NEMB // _L, _zb, 0)
            pltpu.sync_copy(stage_v, sh_hist)

        plsc.subcore_barrier()
        for j in range(nchunk):
            pltpu.sync_copy(ones_v, sh_hist.at[idx_v.at[j]], add=True)
        plsc.subcore_barrier()

        @pl.when(ss == 0)
        def _writeback():
            pltpu.sync_copy(sh_hist, stage_v)

            @pl.when(cc == 0)
            def _w0():
                pltpu.sync_copy(stage_v, hist0_hbm)

            @pl.when(cc == 1)
            def _w1():
                pltpu.sync_copy(stage_v, hist1_hbm)

    return k(emb_pad, idx3d)


HALF = NEMB // 2


def _fin_body(x_ref, qa_ref, qb_ref, ha0_ref, ha1_ref, hb0_ref, hb1_ref,
              q_ref, loss_ref, perp_ref):
    x = x_ref[...]
    qa = qa_ref[:, 0:DIM]
    qb = qb_ref[:, 0:DIM]
    q_ref[0:HALF, :] = qa
    q_ref[HALF:NEMB, :] = qb
    da = x[0:HALF, :] - qa
    db = x[HALF:NEMB, :] - qb
    loss_ref[0, 0] = (jnp.sum(da * da) + jnp.sum(db * db)) / (NEMB * DIM)
    h = (ha0_ref[...] + ha1_ref[...]) + (hb0_ref[...] + hb1_ref[...])
    p = h / float(NEMB)
    ent = jnp.sum(p * jnp.log(p + 1e-10))
    perp_ref[0, 0] = jnp.exp(-ent)


def _fin(x_flat, qa_pad, qb_pad, ha0, ha1, hb0, hb1):
    return pl.pallas_call(
        _fin_body,
        in_specs=[
            pl.BlockSpec((NEMB, DIM), lambda: (0, 0)),
            pl.BlockSpec((HALF, DPAD), lambda: (0, 0)),
            pl.BlockSpec((HALF, DPAD), lambda: (0, 0)),
            pl.BlockSpec((1, NEMB), lambda: (0, 0)),
            pl.BlockSpec((1, NEMB), lambda: (0, 0)),
            pl.BlockSpec((1, NEMB), lambda: (0, 0)),
            pl.BlockSpec((1, NEMB), lambda: (0, 0)),
        ],
        out_specs=[
            pl.BlockSpec((NEMB, DIM), lambda: (0, 0)),
            pl.BlockSpec(memory_space=pltpu.SMEM),
            pl.BlockSpec(memory_space=pltpu.SMEM),
        ],
        out_shape=[
            jax.ShapeDtypeStruct((NEMB, DIM), jnp.float32),
            jax.ShapeDtypeStruct((1, 1), jnp.float32),
            jax.ShapeDtypeStruct((1, 1), jnp.float32),
        ],
    )(x_flat, qa_pad, qb_pad, ha0, ha1, hb0, hb1)


def kernel(x, embedding):
    x_flat = x.reshape(-1, DIM)
    emb_pad = jnp.pad(embedding, ((0, 0), (0, DPAD - DIM)))
    nch = HALF // _NW // _ICH
    idx_a = _dist(x_flat[0:HALF, :], embedding)
    qa_pad, ha0, ha1 = _sc_gather_hist(emb_pad, idx_a.reshape(_NW, nch, _ICH))
    idx_b = _dist(x_flat[HALF:NEMB, :], embedding)
    qb_pad, hb0, hb1 = _sc_gather_hist(emb_pad, idx_b.reshape(_NW, nch, _ICH))
    q, loss, perp = _fin(x_flat, qa_pad, qb_pad,
                         ha0.reshape(1, NEMB), ha1.reshape(1, NEMB),
                         hb0.reshape(1, NEMB), hb1.reshape(1, NEMB))
    return q.reshape(x.shape), loss[0, 0], perp[0, 0]


# speedup vs baseline: 1.0680x; 1.0680x over previous
import functools

import jax
import jax.numpy as jnp
from jax import lax
from jax.experimental import pallas as pl
from jax.experimental.pallas import tpu as pltpu
from jax.experimental.pallas import tpu_sc as plsc

NEMB = 8192
DIM = 32
RB = 1024
CB = 1024
NRB = NEMB // RB
NCB = NEMB // CB

_NC = 2
_NS = 16
_L = 16
_NW = _NC * _NS
_BPW = NEMB // _NW
_ICH = 128
_NCHUNK = _BPW // _ICH

MM_PRECISION = lax.Precision.DEFAULT


def _dist_body(x_ref, emb_ref, idx_ref, embn_ref, crow_ref):
    pid = pl.program_id(0)

    @pl.when(pid == 0)
    def _prep():
        emb = emb_ref[...]
        nrm = jnp.sqrt(jnp.sum(emb * emb, axis=1, keepdims=True))
        emb_n = emb / (nrm + 1e-4)
        embn_ref[...] = emb_n
        crow_ref[...] = lax.dot_general(
            jnp.ones((1, DIM), jnp.float32), emb_n * emb_n,
            (((1,), (1,)), ((), ())),
            preferred_element_type=jnp.float32,
            precision=lax.Precision.HIGHEST)

    x = x_ref[...]
    x2 = jnp.sum(x * x, axis=1, keepdims=True)

    best_val = jnp.full((RB, 1), jnp.inf, dtype=jnp.float32)
    best_idx = jnp.zeros((RB, 1), dtype=jnp.int32)
    for k in range(NCB):
        e_chunk = embn_ref[k * CB:(k + 1) * CB, :]
        mm = lax.dot_general(x, e_chunk, (((1,), (1,)), ((), ())),
                             preferred_element_type=jnp.float32,
                             precision=MM_PRECISION)
        d = (crow_ref[0:1, k * CB:(k + 1) * CB] + x2) - 2.0 * mm
        m = jnp.min(d, axis=1, keepdims=True)
        col = lax.broadcasted_iota(jnp.int32, (RB, CB), 1)
        idx = jnp.min(col, axis=1, keepdims=True,
                      where=(d == m), initial=NEMB) + k * CB
        upd = m < best_val
        best_val = jnp.where(upd, m, best_val)
        best_idx = jnp.where(upd, idx, best_idx)

    idx_ref[...] = best_idx


def _dist(x_flat, embedding):
    nrows = x_flat.shape[0]
    return pl.pallas_call(
        _dist_body,
        grid=(nrows // RB,),
        in_specs=[
            pl.BlockSpec((RB, DIM), lambda i: (i, 0)),
            pl.BlockSpec((NEMB, DIM), lambda i: (0, 0)),
        ],
        out_specs=pl.BlockSpec((RB, 1), lambda i: (i, 0)),
        out_shape=jax.ShapeDtypeStruct((nrows, 1), jnp.int32),
        scratch_shapes=[
            pltpu.VMEM((NEMB, DIM), jnp.float32),
            pltpu.VMEM((1, NEMB), jnp.float32),
        ],
        compiler_params=pltpu.CompilerParams(
            dimension_semantics=("arbitrary",)),
    )(x_flat, embedding)


DPAD = 128


def _sc_gather_hist(emb_pad, idx3d):
    nchunk = idx3d.shape[1]
    bpw = nchunk * _ICH
    nrows = _NW * bpw
    mesh = plsc.VectorSubcoreMesh(core_axis_name="c", subcore_axis_name="s")

    @functools.partial(
        pl.kernel,
        mesh=mesh,
        out_type=[
            jax.ShapeDtypeStruct((nrows, DPAD), jnp.float32),
            jax.ShapeDtypeStruct((NEMB,), jnp.float32),
            jax.ShapeDtypeStruct((NEMB,), jnp.float32),
        ],
        scratch_types=[
            pltpu.VMEM((nchunk, _ICH), jnp.int32),
            pltpu.VMEM((bpw, DPAD), jnp.float32),
            pltpu.VMEM((_ICH,), jnp.float32),
            pltpu.VMEM((NEMB,), jnp.float32),
            pltpu.VMEM_SHARED((NEMB,), jnp.float32),
            pltpu.SemaphoreType.DMA,
        ],
    )
    def k(emb_hbm, idx_hbm, q_hbm, hist0_hbm, hist1_hbm,
          idx_v, rows_v, ones_v, stage_v, sh_hist, sem):
        cc = lax.axis_index("c")
        ss = lax.axis_index("s")
        wid = ss * _NC + cc
        pltpu.sync_copy(idx_hbm.at[wid], idx_v)
        for j in range(nchunk):
            pltpu.async_copy(emb_hbm.at[idx_v.at[j]],
                             rows_v.at[pl.ds(j * _ICH, _ICH)], sem).wait()
        pltpu.sync_copy(rows_v, q_hbm.at[pl.ds(wid * bpw, bpw)])

        def _ones(i, carry):
            ones_v[pl.ds(i * _L, _L)] = jnp.ones((_L,), jnp.float32)
            return carry

        lax.fori_loop(0, _ICH // _L, _ones, 0)

        @pl.when(ss == 0)
        def _zero():
            def _zb(i, carry):
                stage_v[pl.ds(i * _L, _L)] = jnp.zeros((_L,), jnp.float32)
                return carry

            lax.fori_loop(0, NEMB // _L, _zb, 0)
            pltpu.sync_copy(stage_v, sh_hist)

        plsc.subcore_barrier()
        for j in range(nchunk):
            pltpu.sync_copy(ones_v, sh_hist.at[idx_v.at[j]], add=True)
        plsc.subcore_barrier()

        @pl.when(ss == 0)
        def _writeback():
            pltpu.sync_copy(sh_hist, stage_v)

            @pl.when(cc == 0)
            def _w0():
                pltpu.sync_copy(stage_v, hist0_hbm)

            @pl.when(cc == 1)
            def _w1():
                pltpu.sync_copy(stage_v, hist1_hbm)

    return k(emb_pad, idx3d)


HALF = NEMB // 2


def _fin_body(x_ref, qa_ref, qb_ref, ha0_ref, ha1_ref, hb0_ref, hb1_ref,
              q_ref, loss_ref, perp_ref):
    x = x_ref[...]
    qa = qa_ref[:, 0:DIM]
    qb = qb_ref[:, 0:DIM]
    q_ref[0:HALF, :] = qa
    q_ref[HALF:NEMB, :] = qb
    da = x[0:HALF, :] - qa
    db = x[HALF:NEMB, :] - qb
    loss_ref[0, 0] = (jnp.sum(da * da) + jnp.sum(db * db)) / (NEMB * DIM)
    h = (ha0_ref[...] + ha1_ref[...]) + (hb0_ref[...] + hb1_ref[...])
    p = h / float(NEMB)
    ent = jnp.sum(p * jnp.log(p + 1e-10))
    perp_ref[0, 0] = jnp.exp(-ent)


def _fin(x_flat, qa_pad, qb_pad, ha0, ha1, hb0, hb1):
    return pl.pallas_call(
        _fin_body,
        in_specs=[
            pl.BlockSpec((NEMB, DIM), lambda: (0, 0)),
            pl.BlockSpec((HALF, DPAD), lambda: (0, 0)),
            pl.BlockSpec((HALF, DPAD), lambda: (0, 0)),
            pl.BlockSpec((1, NEMB), lambda: (0, 0)),
            pl.BlockSpec((1, NEMB), lambda: (0, 0)),
            pl.BlockSpec((1, NEMB), lambda: (0, 0)),
            pl.BlockSpec((1, NEMB), lambda: (0, 0)),
        ],
        out_specs=[
            pl.BlockSpec((NEMB, DIM), lambda: (0, 0)),
            pl.BlockSpec(memory_space=pltpu.SMEM),
            pl.BlockSpec(memory_space=pltpu.SMEM),
        ],
        out_shape=[
            jax.ShapeDtypeStruct((NEMB, DIM), jnp.float32),
            jax.ShapeDtypeStruct((1, 1), jnp.float32),
            jax.ShapeDtypeStruct((1, 1), jnp.float32),
        ],
    )(x_flat, qa_pad, qb_pad, ha0, ha1, hb0, hb1)


def _fin1_body(x_ref, qp_ref, h0_ref, h1_ref, q_ref, loss_ref, perp_ref):
    x = x_ref[...]
    q = qp_ref[:, 0:DIM]
    q_ref[...] = q
    dd = x - q
    loss_ref[0, 0] = jnp.sum(dd * dd) / (NEMB * DIM)
    h = h0_ref[...] + h1_ref[...]
    p = h / float(NEMB)
    ent = jnp.sum(p * jnp.log(p + 1e-10))
    perp_ref[0, 0] = jnp.exp(-ent)


def _fin1(x_flat, q_pad, h0, h1):
    return pl.pallas_call(
        _fin1_body,
        in_specs=[
            pl.BlockSpec((NEMB, DIM), lambda: (0, 0)),
            pl.BlockSpec((NEMB, DPAD), lambda: (0, 0)),
            pl.BlockSpec((1, NEMB), lambda: (0, 0)),
            pl.BlockSpec((1, NEMB), lambda: (0, 0)),
        ],
        out_specs=[
            pl.BlockSpec((NEMB, DIM), lambda: (0, 0)),
            pl.BlockSpec(memory_space=pltpu.SMEM),
            pl.BlockSpec(memory_space=pltpu.SMEM),
        ],
        out_shape=[
            jax.ShapeDtypeStruct((NEMB, DIM), jnp.float32),
            jax.ShapeDtypeStruct((1, 1), jnp.float32),
            jax.ShapeDtypeStruct((1, 1), jnp.float32),
        ],
    )(x_flat, q_pad, h0, h1)


def kernel(x, embedding):
    x_flat = x.reshape(-1, DIM)
    emb_pad = jnp.pad(embedding, ((0, 0), (0, DPAD - DIM)))
    idx = _dist(x_flat, embedding)
    idx3d = idx.reshape(_NW, _NCHUNK, _ICH)
    q_pad, h0, h1 = _sc_gather_hist(emb_pad, idx3d)
    q, loss, perp = _fin1(x_flat, q_pad,
                          h0.reshape(1, NEMB), h1.reshape(1, NEMB))
    return q.reshape(x.shape), loss[0, 0], perp[0, 0]


# speedup vs baseline: 1.1097x; 1.0391x over previous
import functools

import jax
import jax.numpy as jnp
from jax import lax
from jax.experimental import pallas as pl
from jax.experimental.pallas import tpu as pltpu
from jax.experimental.pallas import tpu_sc as plsc

NEMB = 8192
DIM = 32
RB = 2048
CB = 1024
NRB = NEMB // RB
NCB = NEMB // CB

_NC = 2
_NS = 16
_L = 16
_NW = _NC * _NS
_BPW = NEMB // _NW
_ICH = 128
_NCHUNK = _BPW // _ICH

MM_PRECISION = lax.Precision.DEFAULT


def _dist_body(x_ref, emb_ref, idx_ref, embn_ref, crow_ref):
    pid = pl.program_id(0)

    @pl.when(pid == 0)
    def _prep():
        emb = emb_ref[...]
        nrm = jnp.sqrt(jnp.sum(emb * emb, axis=1, keepdims=True))
        emb_n = emb / (nrm + 1e-4)
        embn_ref[...] = emb_n
        crow_ref[...] = lax.dot_general(
            jnp.ones((1, DIM), jnp.float32), emb_n * emb_n,
            (((1,), (1,)), ((), ())),
            preferred_element_type=jnp.float32,
            precision=lax.Precision.HIGHEST)

    x = x_ref[...]
    x2 = jnp.sum(x * x, axis=1, keepdims=True)

    best_val = jnp.full((RB, 1), jnp.inf, dtype=jnp.float32)
    best_idx = jnp.zeros((RB, 1), dtype=jnp.int32)
    for k in range(NCB):
        e_chunk = embn_ref[k * CB:(k + 1) * CB, :]
        mm = lax.dot_general(x, e_chunk, (((1,), (1,)), ((), ())),
                             preferred_element_type=jnp.float32,
                             precision=MM_PRECISION)
        d = (crow_ref[0:1, k * CB:(k + 1) * CB] + x2) - 2.0 * mm
        m = jnp.min(d, axis=1, keepdims=True)
        col = lax.broadcasted_iota(jnp.int32, (RB, CB), 1)
        idx = jnp.min(jnp.where(d == m, col, NEMB), axis=1,
                      keepdims=True) + k * CB
        upd = m < best_val
        best_val = jnp.where(upd, m, best_val)
        best_idx = jnp.where(upd, idx, best_idx)

    idx_ref[...] = best_idx


def _dist(x_flat, embedding):
    nrows = x_flat.shape[0]
    return pl.pallas_call(
        _dist_body,
        grid=(nrows // RB,),
        in_specs=[
            pl.BlockSpec((RB, DIM), lambda i: (i, 0)),
            pl.BlockSpec((NEMB, DIM), lambda i: (0, 0)),
        ],
        out_specs=pl.BlockSpec((RB, 1), lambda i: (i, 0)),
        out_shape=jax.ShapeDtypeStruct((nrows, 1), jnp.int32),
        scratch_shapes=[
            pltpu.VMEM((NEMB, DIM), jnp.float32),
            pltpu.VMEM((1, NEMB), jnp.float32),
        ],
        compiler_params=pltpu.CompilerParams(
            dimension_semantics=("arbitrary",)),
    )(x_flat, embedding)


DPAD = 128


def _sc_gather_hist(emb_pad, idx3d):
    nchunk = idx3d.shape[1]
    bpw = nchunk * _ICH
    nrows = _NW * bpw
    mesh = plsc.VectorSubcoreMesh(core_axis_name="c", subcore_axis_name="s")

    @functools.partial(
        pl.kernel,
        mesh=mesh,
        out_type=[
            jax.ShapeDtypeStruct((nrows, DPAD), jnp.float32),
            jax.ShapeDtypeStruct((NEMB,), jnp.float32),
            jax.ShapeDtypeStruct((NEMB,), jnp.float32),
        ],
        scratch_types=[
            pltpu.VMEM((nchunk, _ICH), jnp.int32),
            pltpu.VMEM((bpw, DPAD), jnp.float32),
            pltpu.VMEM((_ICH,), jnp.float32),
            pltpu.VMEM((NEMB,), jnp.float32),
            pltpu.VMEM_SHARED((NEMB,), jnp.float32),
            pltpu.SemaphoreType.DMA,
        ],
    )
    def k(emb_hbm, idx_hbm, q_hbm, hist0_hbm, hist1_hbm,
          idx_v, rows_v, ones_v, stage_v, sh_hist, sem):
        cc = lax.axis_index("c")
        ss = lax.axis_index("s")
        wid = ss * _NC + cc
        pltpu.sync_copy(idx_hbm.at[wid], idx_v)
        for j in range(nchunk):
            pltpu.async_copy(emb_hbm.at[idx_v.at[j]],
                             rows_v.at[pl.ds(j * _ICH, _ICH)], sem).wait()
        pltpu.sync_copy(rows_v, q_hbm.at[pl.ds(wid * bpw, bpw)])

        def _ones(i, carry):
            ones_v[pl.ds(i * _L, _L)] = jnp.ones((_L,), jnp.float32)
            return carry

        lax.fori_loop(0, _ICH // _L, _ones, 0)

        @pl.when(ss == 0)
        def _zero():
            def _zb(i, carry):
                stage_v[pl.ds(i * _L, _L)] = jnp.zeros((_L,), jnp.float32)
                return carry

            lax.fori_loop(0, NEMB // _L, _zb, 0)
            pltpu.sync_copy(stage_v, sh_hist)

        plsc.subcore_barrier()
        for j in range(nchunk):
            pltpu.sync_copy(ones_v, sh_hist.at[idx_v.at[j]], add=True)
        plsc.subcore_barrier()

        @pl.when(ss == 0)
        def _writeback():
            pltpu.sync_copy(sh_hist, stage_v)

            @pl.when(cc == 0)
            def _w0():
                pltpu.sync_copy(stage_v, hist0_hbm)

            @pl.when(cc == 1)
            def _w1():
                pltpu.sync_copy(stage_v, hist1_hbm)

    return k(emb_pad, idx3d)


HALF = NEMB // 2


def _fin_body(x_ref, qa_ref, qb_ref, ha0_ref, ha1_ref, hb0_ref, hb1_ref,
              q_ref, loss_ref, perp_ref):
    x = x_ref[...]
    qa = qa_ref[:, 0:DIM]
    qb = qb_ref[:, 0:DIM]
    q_ref[0:HALF, :] = qa
    q_ref[HALF:NEMB, :] = qb
    da = x[0:HALF, :] - qa
    db = x[HALF:NEMB, :] - qb
    loss_ref[0, 0] = (jnp.sum(da * da) + jnp.sum(db * db)) / (NEMB * DIM)
    h = (ha0_ref[...] + ha1_ref[...]) + (hb0_ref[...] + hb1_ref[...])
    p = h / float(NEMB)
    ent = jnp.sum(p * jnp.log(p + 1e-10))
    perp_ref[0, 0] = jnp.exp(-ent)


def _fin(x_flat, qa_pad, qb_pad, ha0, ha1, hb0, hb1):
    return pl.pallas_call(
        _fin_body,
        in_specs=[
            pl.BlockSpec((NEMB, DIM), lambda: (0, 0)),
            pl.BlockSpec((HALF, DPAD), lambda: (0, 0)),
            pl.BlockSpec((HALF, DPAD), lambda: (0, 0)),
            pl.BlockSpec((1, NEMB), lambda: (0, 0)),
            pl.BlockSpec((1, NEMB), lambda: (0, 0)),
            pl.BlockSpec((1, NEMB), lambda: (0, 0)),
            pl.BlockSpec((1, NEMB), lambda: (0, 0)),
        ],
        out_specs=[
            pl.BlockSpec((NEMB, DIM), lambda: (0, 0)),
            pl.BlockSpec(memory_space=pltpu.SMEM),
            pl.BlockSpec(memory_space=pltpu.SMEM),
        ],
        out_shape=[
            jax.ShapeDtypeStruct((NEMB, DIM), jnp.float32),
            jax.ShapeDtypeStruct((1, 1), jnp.float32),
            jax.ShapeDtypeStruct((1, 1), jnp.float32),
        ],
    )(x_flat, qa_pad, qb_pad, ha0, ha1, hb0, hb1)


def _fin1_body(x_ref, qp_ref, h0_ref, h1_ref, q_ref, loss_ref, perp_ref):
    x = x_ref[...]
    q = qp_ref[:, 0:DIM]
    q_ref[...] = q
    dd = x - q
    loss_ref[0, 0] = jnp.sum(dd * dd) / (NEMB * DIM)
    h = h0_ref[...] + h1_ref[...]
    p = h / float(NEMB)
    ent = jnp.sum(p * jnp.log(p + 1e-10))
    perp_ref[0, 0] = jnp.exp(-ent)


def _fin1(x_flat, q_pad, h0, h1):
    return pl.pallas_call(
        _fin1_body,
        in_specs=[
            pl.BlockSpec((NEMB, DIM), lambda: (0, 0)),
            pl.BlockSpec((NEMB, DPAD), lambda: (0, 0)),
            pl.BlockSpec((1, NEMB), lambda: (0, 0)),
            pl.BlockSpec((1, NEMB), lambda: (0, 0)),
        ],
        out_specs=[
            pl.BlockSpec((NEMB, DIM), lambda: (0, 0)),
            pl.BlockSpec(memory_space=pltpu.SMEM),
            pl.BlockSpec(memory_space=pltpu.SMEM),
        ],
        out_shape=[
            jax.ShapeDtypeStruct((NEMB, DIM), jnp.float32),
            jax.ShapeDtypeStruct((1, 1), jnp.float32),
            jax.ShapeDtypeStruct((1, 1), jnp.float32),
        ],
    )(x_flat, q_pad, h0, h1)


def kernel(x, embedding):
    x_flat = x.reshape(-1, DIM)
    emb_pad = jnp.pad(embedding, ((0, 0), (0, DPAD - DIM)))
    idx = _dist(x_flat, embedding)
    idx3d = idx.reshape(_NW, _NCHUNK, _ICH)
    q_pad, h0, h1 = _sc_gather_hist(emb_pad, idx3d)
    q, loss, perp = _fin1(x_flat, q_pad,
                          h0.reshape(1, NEMB), h1.reshape(1, NEMB))
    return q.reshape(x.shape), loss[0, 0], perp[0, 0]
